# Initial kernel scaffold; baseline (speedup 1.0000x reference)
import jax
import jax.numpy as jnp
from jax.experimental import pallas as pl


def kernel(ptc, ftc, w1, b1, w2):
    raise NotImplementedError("write your pallas kernel here")


# speedup vs baseline: 19.0084x; 19.0084x over previous
import functools

import jax
import jax.numpy as jnp
import numpy as np
from jax import lax
from jax.experimental import pallas as pl
from jax.experimental.pallas import tpu as pltpu
from jax.experimental.pallas import tpu_sc as plsc

_T = 50
_H = 8
_W = 8
_NH = 16
_RPE = 16.0
_RCT = 8.0
_ND = 2 * _T - 1
_NS = 225
_NSP = 256


def _static_tables():
    rt = np.arange(0, 2 * _T - 1, dtype=np.float32)
    rh = np.arange(-_H + 1, _H, dtype=np.float32)
    rw = np.arange(-_W + 1, _W, dtype=np.float32)
    tab = np.stack(np.meshgrid(rt, rh, rw, indexing='ij'), axis=0)
    tab = tab.transpose(1, 2, 3, 0).copy()
    tab[..., 0] /= (2 * _T - 1)
    tab[..., 1] /= (_H - 1)
    tab[..., 2] /= (_W - 1)
    tab = np.sign(tab) * np.log2(np.abs(tab) * _RCT + 1.0) / np.log2(_RCT)
    x = tab.reshape(-1, 3)

    xT = np.zeros((_ND, 8, _NSP), dtype=np.float32)
    for d in range(_ND):
        xT[d, :3, :_NS] = x[d * _NS:(d + 1) * _NS].T
        xT[d, 3, :_NS] = 1.0

    s = np.arange(64)
    h1, w1 = s // 8, s % 8
    k = (h1[:, None] - h1[None, :] + 7) * 15 + (w1[:, None] - w1[None, :] + 7)
    gt = np.zeros((_NSP, 64 * 64), dtype=np.float32)
    gt[k.reshape(-1), np.arange(64 * 64)] = 1.0
    return xT, gt


_XT, _GT = _static_tables()


def _blk_body(xt_ref, w1t_ref, w2t_ref, gt_ref, out_ref):
    hdn = jnp.dot(w1t_ref[...], xt_ref[0],
                  preferred_element_type=jnp.float32)
    hdn = jnp.maximum(hdn, 0.0)
    rptt = jnp.dot(w2t_ref[...], hdn,
                   preferred_element_type=jnp.float32)
    blkt = jnp.dot(rptt, gt_ref[...],
                   preferred_element_type=jnp.float32)
    out_ref[0] = _RPE * jax.nn.sigmoid(blkt)


def _build_blk(w1t, w2t):
    return pl.pallas_call(
        _blk_body,
        grid=(_ND,),
        in_specs=[
            pl.BlockSpec((1, 8, _NSP), lambda d: (d, 0, 0)),
            pl.BlockSpec((512, 8), lambda d: (0, 0)),
            pl.BlockSpec((_NH, 512), lambda d: (0, 0)),
            pl.BlockSpec((_NSP, 64 * 64), lambda d: (0, 0)),
        ],
        out_specs=pl.BlockSpec((1, _NH, 64 * 64), lambda d: (d, 0, 0)),
        out_shape=jax.ShapeDtypeStruct((_ND, _NH, 64 * 64), jnp.float32),
    )(_XT, w1t, w2t, _GT)


def _make_sc_copy(B, TF, TP):
    mesh = plsc.VectorSubcoreMesh(core_axis_name="c", subcore_axis_name="s")

    @functools.partial(
        pl.kernel,
        mesh=mesh,
        compiler_params=pltpu.CompilerParams(
            use_tc_tiling_on_sc=False, needs_layout_passes=False),
        out_type=jax.ShapeDtypeStruct((B, _NH, TF * 64, TP * 64), jnp.float32),
        scratch_types=[
            pltpu.VMEM((16,), jnp.int32),
            pltpu.VMEM((16,), jnp.int32),
            pltpu.VMEM((_NH, 64, 64), jnp.float32),
        ],
    )
    def sc_copy(blk_hbm, ptc_hbm, ftc_hbm, out_hbm, ptc_v, ftc_v, buf):
        b = lax.axis_index("c")
        fi = lax.axis_index("s")
        pltpu.sync_copy(ptc_hbm.at[b], ptc_v)
        pltpu.sync_copy(ftc_hbm.at[b], ftc_v)
        lanes = lax.iota(jnp.int32, 16)
        fts = jnp.sum(jnp.where(lanes == fi, ftc_v[...], 0))
        d_vec = fts - ptc_v[...]
        for j in range(TP):
            dj = jnp.sum(jnp.where(lanes == j, d_vec, 0))
            pltpu.sync_copy(blk_hbm.at[dj], buf)
            pltpu.sync_copy(
                buf,
                out_hbm.at[b, :, pl.ds(fi * 64, 64), pl.ds(j * 64, 64)])

    return sc_copy


def kernel(ptc, ftc, w1, b1, w2):
    B, TP = ptc.shape
    TF = ftc.shape[1]
    w1aug = jnp.concatenate([w1, b1[None, :]], axis=0)
    w1t = jnp.zeros((512, 8), jnp.float32).at[:, :4].set(w1aug.T)
    w2t = w2.T
    blk = _build_blk(w1t, w2t).reshape(_ND, _NH, 64, 64)
    return _make_sc_copy(B, TF, TP)(blk, ptc, ftc)


# speedup vs baseline: 20.9941x; 1.1045x over previous
import functools

import jax
import jax.numpy as jnp
import numpy as np
from jax import lax
from jax.experimental import pallas as pl
from jax.experimental.pallas import tpu as pltpu
from jax.experimental.pallas import tpu_sc as plsc

_T = 50
_H = 8
_W = 8
_NH = 16
_RPE = 16.0
_RCT = 8.0
_ND = 2 * _T - 1
_NDP = 104
_DPB = 8
_NS = 225
_NSP = 256


def _static_tables():
    rt = np.arange(0, 2 * _T - 1, dtype=np.float32)
    rh = np.arange(-_H + 1, _H, dtype=np.float32)
    rw = np.arange(-_W + 1, _W, dtype=np.float32)
    tab = np.stack(np.meshgrid(rt, rh, rw, indexing='ij'), axis=0)
    tab = tab.transpose(1, 2, 3, 0).copy()
    tab[..., 0] /= (2 * _T - 1)
    tab[..., 1] /= (_H - 1)
    tab[..., 2] /= (_W - 1)
    tab = np.sign(tab) * np.log2(np.abs(tab) * _RCT + 1.0) / np.log2(_RCT)
    x = tab.reshape(-1, 3)

    xT = np.zeros((_NDP, 8, _NSP), dtype=np.float32)
    for d in range(_ND):
        xT[d, :3, :_NS] = x[d * _NS:(d + 1) * _NS].T
        xT[d, 3, :_NS] = 1.0

    s = np.arange(64)
    h1, w1 = s // 8, s % 8
    k = (h1[:, None] - h1[None, :] + 7) * 15 + (w1[:, None] - w1[None, :] + 7)
    gt = np.zeros((_NSP, 64 * 64), dtype=np.float32)
    gt[k.reshape(-1), np.arange(64 * 64)] = 1.0
    return xT, gt


_XT, _GT = _static_tables()


def _blk_body(xt_ref, w1t_ref, w2t_ref, gt_ref, out_ref):
    rows = []
    for i in range(_DPB):
        hdn = jnp.dot(w1t_ref[...], xt_ref[i],
                      preferred_element_type=jnp.float32)
        hdn = jnp.maximum(hdn, 0.0)
        rows.append(jnp.dot(w2t_ref[...], hdn,
                            preferred_element_type=jnp.float32))
    rptt = jnp.concatenate(rows, axis=0)
    blkt = jnp.dot(rptt, gt_ref[...],
                   preferred_element_type=jnp.float32)
    out_ref[...] = (_RPE * jax.nn.sigmoid(blkt)).reshape(_DPB, _NH, 64 * 64)


def _build_blk(w1t, w2t):
    return pl.pallas_call(
        _blk_body,
        grid=(_NDP // _DPB,),
        in_specs=[
            pl.BlockSpec((_DPB, 8, _NSP), lambda t: (t, 0, 0)),
            pl.BlockSpec((512, 8), lambda t: (0, 0)),
            pl.BlockSpec((_NH, 512), lambda t: (0, 0)),
            pl.BlockSpec((_NSP, 64 * 64), lambda t: (0, 0)),
        ],
        out_specs=pl.BlockSpec((_DPB, _NH, 64 * 64), lambda t: (t, 0, 0)),
        out_shape=jax.ShapeDtypeStruct((_NDP, _NH, 64 * 64), jnp.float32),
    )(_XT, w1t, w2t, _GT)


def _make_sc_copy(B, TF, TP):
    mesh = plsc.VectorSubcoreMesh(core_axis_name="c", subcore_axis_name="s")
    n_units = TP * 2

    @functools.partial(
        pl.kernel,
        mesh=mesh,
        compiler_params=pltpu.CompilerParams(
            use_tc_tiling_on_sc=False, needs_layout_passes=False),
        out_type=jax.ShapeDtypeStruct((B, _NH, TF * 64, TP * 64), jnp.float32),
        scratch_types=[
            pltpu.VMEM((16,), jnp.int32),
            pltpu.VMEM((16,), jnp.int32),
            pltpu.VMEM((8, 64, 64), jnp.float32),
            pltpu.VMEM((8, 64, 64), jnp.float32),
            pltpu.VMEM((8, 64, 64), jnp.float32),
            pltpu.SemaphoreType.DMA,
            pltpu.SemaphoreType.DMA,
            pltpu.SemaphoreType.DMA,
            pltpu.SemaphoreType.DMA,
            pltpu.SemaphoreType.DMA,
            pltpu.SemaphoreType.DMA,
        ],
    )
    def sc_copy(blk_hbm, ptc_hbm, ftc_hbm, out_hbm, ptc_v, ftc_v,
                buf0, buf1, buf2, gs0, gs1, gs2, ss0, ss1, ss2):
        bufs = (buf0, buf1, buf2)
        gsems = (gs0, gs1, gs2)
        ssems = (ss0, ss1, ss2)
        b = lax.axis_index("c")
        fi = lax.axis_index("s")
        pltpu.sync_copy(ptc_hbm.at[b], ptc_v)
        pltpu.sync_copy(ftc_hbm.at[b], ftc_v)
        lanes = lax.iota(jnp.int32, 16)
        fts = jnp.sum(jnp.where(lanes == fi, ftc_v[...], 0))
        d_vec = fts - ptc_v[...]

        def src(u):
            j, c = u // 2, (u % 2) * 8
            dj = jnp.sum(jnp.where(lanes == j, d_vec, 0))
            return blk_hbm.at[dj, pl.ds(c, 8)]

        def dst(u):
            j, c = u // 2, (u % 2) * 8
            return out_hbm.at[b, pl.ds(c, 8),
                              pl.ds(fi * 64, 64), pl.ds(j * 64, 64)]

        gh = [None] * n_units
        sh = [None] * n_units
        gh[0] = pltpu.async_copy(src(0), bufs[0], gsems[0])
        for u in range(n_units):
            k = u % 3
            if u + 1 < n_units:
                k1 = (u + 1) % 3
                if u + 1 >= 3:
                    sh[u - 2].wait()
                gh[u + 1] = pltpu.async_copy(src(u + 1), bufs[k1], gsems[k1])
            gh[u].wait()
            sh[u] = pltpu.async_copy(bufs[k], dst(u), ssems[k])
        for u in range(n_units - 3, n_units):
            sh[u].wait()

    return sc_copy


def kernel(ptc, ftc, w1, b1, w2):
    B, TP = ptc.shape
    TF = ftc.shape[1]
    w1aug = jnp.concatenate([w1, b1[None, :]], axis=0)
    w1t = jnp.zeros((512, 8), jnp.float32).at[:, :4].set(w1aug.T)
    w2t = w2.T
    blk = _build_blk(w1t, w2t).reshape(_NDP, _NH, 64, 64)
    return _make_sc_copy(B, TF, TP)(blk, ptc, ftc)
